# baseline (speedup 1.0000x reference)
import functools

import jax
import jax.numpy as jnp
from jax import lax
from jax.experimental import pallas as pl
from jax.experimental.pallas import tpu as pltpu
from jax.experimental.pallas import tpu_sc as plsc

_NC = 2
_NS = 16
_LANES = 16
_NW = _NC * _NS
_VPAD = 128


def _sc_counts(indices):
    B, L = indices.shape
    bags_per_w = B // _NW
    n_chunks = 8
    chunk_b = bags_per_w // n_chunks
    groups_per_chunk = chunk_b // _LANES

    mesh = plsc.VectorSubcoreMesh(
        core_axis_name="c", subcore_axis_name="s",
        num_cores=_NC, num_subcores=_NS)

    @functools.partial(
        pl.kernel,
        mesh=mesh,
        out_type=jax.ShapeDtypeStruct((B, _VPAD), jnp.float32),
        scratch_types=[
            pltpu.VMEM((chunk_b, L), jnp.int32),
            pltpu.VMEM((chunk_b, L), jnp.int32),
            pltpu.VMEM((bags_per_w, _VPAD), jnp.float32),
            pltpu.SemaphoreType.DMA,
            pltpu.SemaphoreType.DMA,
        ],
        compiler_params=pltpu.CompilerParams(
            needs_layout_passes=False, use_tc_tiling_on_sc=True),
    )
    def counts_kernel(idx_hbm, counts_hbm, idx_v0, idx_v1, cnt_v, sem0, sem1):
        wid = lax.axis_index("s") * _NC + lax.axis_index("c")
        base = wid * bags_per_w
        bufs = (idx_v0, idx_v1)
        sems = (sem0, sem1)

        def start_load(c):
            return pltpu.async_copy(
                idx_hbm.at[pl.ds(base + c * chunk_b, chunk_b), :],
                bufs[c % 2], sems[c % 2])

        cps = [start_load(0), start_load(1)]

        rows16 = lax.iota(jnp.int32, 16)
        ones16 = jnp.ones((16,), jnp.float32)
        zeros16 = jnp.zeros((16,), jnp.float32)

        @plsc.parallel_loop(0, bags_per_w, unroll=2)
        def _zero(r):
            for c in range(_VPAD // 16):
                cnt_v[r, pl.ds(c * 16, 16)] = zeros16

        for chunk in range(n_chunks):
            cps[chunk].wait()
            idx_v = bufs[chunk % 2]

            def group_body(g, carry, idx_v=idx_v, chunk=chunk):
                grows = g * _LANES + rows16
                crows = chunk * chunk_b + grows

                @plsc.parallel_loop(0, L, unroll=8)
                def _accum(l):
                    col = jnp.full((16,), l, jnp.int32)
                    idxv = plsc.load_gather(idx_v, [grows, col])
                    plsc.addupdate_scatter(cnt_v, [crows, idxv], ones16)

                return carry

            lax.fori_loop(0, groups_per_chunk, group_body, 0)
            if chunk + 2 < n_chunks:
                cps.append(start_load(chunk + 2))

        pltpu.sync_copy(cnt_v, counts_hbm.at[pl.ds(base, bags_per_w), :])

    return counts_kernel(indices)


def _tc_combine(counts, table_p, inv_l):
    B = counts.shape[0]
    D = table_p.shape[1]
    blk = 2048

    def mm(cnt_ref, tab_ref, o_ref):
        o_ref[...] = jnp.dot(
            cnt_ref[...], tab_ref[...],
            preferred_element_type=jnp.float32) * inv_l

    return pl.pallas_call(
        mm,
        grid=(B // blk,),
        in_specs=[
            pl.BlockSpec((blk, _VPAD), lambda i: (i, 0)),
            pl.BlockSpec((_VPAD, D), lambda i: (0, 0)),
        ],
        out_specs=pl.BlockSpec((blk, D), lambda i: (i, 0)),
        out_shape=jax.ShapeDtypeStruct((B, D), jnp.float32),
    )(counts, table_p)


def kernel(indices, table):
    _, L = indices.shape
    V, D = table.shape
    counts = _sc_counts(indices.astype(jnp.int32))
    table_p = jnp.zeros((_VPAD, D), table.dtype).at[:V, :].set(table)
    return _tc_combine(counts, table_p, 1.0 / L)


# speedup vs baseline: 1.1480x; 1.1480x over previous
import functools

import jax
import jax.numpy as jnp
from jax import lax
from jax.experimental import pallas as pl
from jax.experimental.pallas import tpu as pltpu
from jax.experimental.pallas import tpu_sc as plsc

_NC = 2
_NS = 16
_LANES = 16
_NW = _NC * _NS
_VPAD = 128


def _sc_counts(indices_flat, B, L):
    bags_per_w = B // _NW
    n_chunks = 4
    chunk_b = bags_per_w // n_chunks
    groups_per_chunk = chunk_b // _LANES

    mesh = plsc.VectorSubcoreMesh(
        core_axis_name="c", subcore_axis_name="s",
        num_cores=_NC, num_subcores=_NS)

    @functools.partial(
        pl.kernel,
        mesh=mesh,
        out_type=jax.ShapeDtypeStruct((B, _VPAD), jnp.float32),
        scratch_types=[
            pltpu.VMEM((chunk_b * L,), jnp.int32),
            pltpu.VMEM((chunk_b * L,), jnp.int32),
            pltpu.VMEM((bags_per_w, _VPAD), jnp.float32),
            pltpu.SemaphoreType.DMA,
            pltpu.SemaphoreType.DMA,
        ],
        compiler_params=pltpu.CompilerParams(
            needs_layout_passes=False, use_tc_tiling_on_sc=False),
    )
    def counts_kernel(idx_hbm, counts_hbm, idx_v0, idx_v1, cnt_v, sem0, sem1):
        wid = lax.axis_index("s") * _NC + lax.axis_index("c")
        base = wid * bags_per_w
        bufs = (idx_v0, idx_v1)
        sems = (sem0, sem1)

        def start_load(c):
            return pltpu.async_copy(
                idx_hbm.at[pl.ds((base + c * chunk_b) * L, chunk_b * L)],
                bufs[c % 2], sems[c % 2])

        cps = [start_load(0), start_load(1)]

        rows16 = lax.iota(jnp.int32, 16)
        ones16 = jnp.ones((16,), jnp.float32)
        zeros16 = jnp.zeros((16,), jnp.float32)

        @plsc.parallel_loop(0, bags_per_w, unroll=2)
        def _zero(r):
            for c in range(_VPAD // 16):
                cnt_v[r, pl.ds(c * 16, 16)] = zeros16

        for chunk in range(n_chunks):
            cps[chunk].wait()
            idx_v = bufs[chunk % 2]

            def group_body(g, carry, idx_v=idx_v, chunk=chunk):
                grows = g * _LANES + rows16
                gbase = grows * L
                crows = chunk * chunk_b + grows

                @plsc.parallel_loop(0, L, unroll=8)
                def _accum(l):
                    idxv = plsc.load_gather(idx_v, [gbase + l])
                    plsc.addupdate_scatter(cnt_v, [crows, idxv], ones16)

                return carry

            lax.fori_loop(0, groups_per_chunk, group_body, 0)
            if chunk + 2 < n_chunks:
                cps.append(start_load(chunk + 2))

        pltpu.sync_copy(cnt_v, counts_hbm.at[pl.ds(base, bags_per_w), :])

    return counts_kernel(indices_flat)


def _tc_combine(counts, table_p, inv_l):
    B = counts.shape[0]
    D = table_p.shape[1]
    blk = 2048

    def mm(cnt_ref, tab_ref, o_ref):
        o_ref[...] = jnp.dot(
            cnt_ref[...], tab_ref[...],
            preferred_element_type=jnp.float32) * inv_l

    return pl.pallas_call(
        mm,
        grid=(B // blk,),
        in_specs=[
            pl.BlockSpec((blk, _VPAD), lambda i: (i, 0)),
            pl.BlockSpec((_VPAD, D), lambda i: (0, 0)),
        ],
        out_specs=pl.BlockSpec((blk, D), lambda i: (i, 0)),
        out_shape=jax.ShapeDtypeStruct((B, D), jnp.float32),
    )(counts, table_p)


def kernel(indices, table):
    B, L = indices.shape
    V, D = table.shape
    idx128 = indices.astype(jnp.int32).reshape(B * L // 128, 128)
    idx_flat = jax.lax.optimization_barrier(idx128).reshape(B * L)
    counts = _sc_counts(idx_flat, B, L)
    table_p = jnp.zeros((_VPAD, D), table.dtype).at[:V, :].set(table)
    return _tc_combine(counts, table_p, 1.0 / L)
